# Initial kernel scaffold; baseline (speedup 1.0000x reference)
import jax
import jax.numpy as jnp
from jax.experimental import pallas as pl


def kernel(d0, d1, d2, d3, d4, r_glass, r_global, ground_truth):
    raise NotImplementedError("write your pallas kernel here")


# speedup vs baseline: 3.9210x; 3.9210x over previous
import jax
import jax.numpy as jnp
from jax import lax
from jax.experimental import pallas as pl
from jax.experimental.pallas import tpu as pltpu
from jax.experimental.pallas import tpu_sc as plsc

B, H, W = 4, 512, 512
N = H * W
NPAIR = 20
NTILE = 16
CHUNK = N // NTILE
SUB = CHUNK // 16
RAD = 256
NC_TC = 8
C_TC = N // NC_TC
import numpy as np
MIN32 = np.int32(-2147483648)


def _prep_body(d0, d1, d2, d3, d4, gt, rgl, rgo, kall, msep):
    gti = gt[...]
    gtf = gti.astype(jnp.float32)
    sign = 2.0 * gtf - 1.0
    for dd, dref in enumerate((d0, d1, d2, d3, d4)):
        x = dref[...]
        e = 1.0 - x * sign
        bi = lax.bitcast_convert_type(e, jnp.int32)
        srt = jnp.where(bi < 0, jnp.bitwise_xor(bi, -1),
                        jnp.bitwise_xor(bi, MIN32))
        k = jnp.bitwise_xor(srt, -1)
        k = jnp.bitwise_or(jnp.bitwise_and(k, -2), gti)
        kall[dd] = k
    g3 = gtf[:, None]
    df = (rgo[...] - rgl[...]) * g3
    msep[0, 0, 0] = jnp.sum(df * df)


def _sc_body(keys_hbm, out_hbm, bufA, bufB, gT, gLab, gRed,
             tk, hist, bases, gtl, glabv, gredv, posb, stg, stgi, stgf):
    c = lax.axis_index("c")
    s = lax.axis_index("s")
    iota = lax.iota(jnp.int32, 16)
    ones = jnp.ones((16,), jnp.int32)
    zer16 = jnp.zeros((16,), jnp.int32)

    def do_pair(jp, carry_outer):
        p = 2 * jp + c
        base_off = p * N

        for pass_i in range(4):
            if pass_i == 0:
                off = pl.multiple_of(base_off + s * CHUNK, CHUNK)
                pltpu.sync_copy(keys_hbm.at[pl.ds(off, CHUNK)], tk)
                dst = bufB
            else:
                src = bufB if pass_i % 2 == 1 else bufA
                dst = bufA if pass_i % 2 == 1 else bufB
                off = pl.multiple_of(s * CHUNK, CHUNK)
                pltpu.sync_copy(src.at[pl.ds(off, CHUNK)], tk)
            shift = jnp.full((16,), 8 * pass_i, jnp.int32)

            def zb(i, _):
                plsc.store_scatter(hist, [i * 16 + iota], zer16)
                return 0
            lax.fori_loop(0, RAD, zb, 0)

            def hb(jj, _):
                idxv = iota * SUB + jj
                kv = plsc.load_gather(tk, [idxv])
                dg = jnp.bitwise_and(lax.shift_right_logical(kv, shift), 255)
                plsc.addupdate_scatter(hist, [dg * 16 + iota], ones)
                return 0
            lax.fori_loop(0, SUB, hb, 0)

            def mg(g, _):
                dd = g * 16 + iota
                acc = zer16
                for l in range(16):
                    acc = acc + plsc.load_gather(hist, [dd * 16 + l])
                plsc.store_scatter(stg, [g * 16 + iota], acc)
                return 0
            lax.fori_loop(0, 16, mg, 0)
            pltpu.sync_copy(stg, gT.at[s])
            plsc.subcore_barrier()
            pltpu.sync_copy(gT, gtl)

            def sb(d, carry):
                dvec = jnp.full((16,), d, jnp.int32)
                tvec = plsc.load_gather(gtl, [iota, dvec])
                tinc = plsc.cumsum(tvec)
                tex_t = jnp.sum(jnp.where(iota == s, tinc - tvec, 0))
                hvec = plsc.load_gather(hist, [d * 16 + iota])
                hex_ = plsc.cumsum(hvec) - hvec
                plsc.store_scatter(bases, [d * 16 + iota],
                                   carry + tex_t + hex_)
                return carry + jnp.sum(tvec)
            lax.fori_loop(0, RAD, sb, jnp.int32(0))

            def pb(jj, _):
                idxv = iota * SUB + jj
                kv = plsc.load_gather(tk, [idxv])
                dg = jnp.bitwise_and(lax.shift_right_logical(kv, shift), 255)
                fl = dg * 16 + iota
                pos = plsc.load_gather(bases, [fl])
                plsc.store_scatter(bases, [fl], pos + ones)
                plsc.store_scatter(posb, [idxv], pos)
                return 0
            lax.fori_loop(0, SUB, pb, 0)
            pltpu.sync_copy(tk, dst.at[posb])
            plsc.subcore_barrier()

        off = pl.multiple_of(s * CHUNK, CHUNK)
        pltpu.sync_copy(bufA.at[pl.ds(off, CHUNK)], tk)

        def l1(jj, acc):
            kv = plsc.load_gather(tk, [jj * 16 + iota])
            return acc + jnp.bitwise_and(kv, 1)
        accv = lax.fori_loop(0, SUB, l1, zer16)
        tot = jnp.sum(accv)
        stgi[...] = jnp.full((16,), tot, jnp.int32)
        pltpu.sync_copy(stgi, gLab.at[s])
        plsc.subcore_barrier()
        pltpu.sync_copy(gLab, glabv)
        totv = plsc.load_gather(glabv, [iota, zer16])
        pcount = jnp.sum(totv)
        cbase = jnp.sum(jnp.where(iota == s, plsc.cumsum(totv) - totv, 0))
        pf = pcount.astype(jnp.float32)

        def l2(jj, carry):
            c_run, lacc = carry
            kv = plsc.load_gather(tk, [jj * 16 + iota])
            lab = jnp.bitwise_and(kv, 1)
            s2 = jnp.bitwise_xor(kv, -1)
            u2 = jnp.where(s2 < 0, jnp.bitwise_xor(s2, MIN32),
                           jnp.bitwise_xor(s2, -1))
            e = plsc.bitcast(u2, jnp.float32)
            relu = jnp.maximum(e, 0.0)
            cvec = (c_run + cbase + plsc.cumsum(lab)).astype(jnp.float32)
            ivec = (s * CHUNK + jj * 16 + iota).astype(jnp.float32)
            t2 = ivec + pf - cvec
            gpos = 1.0 / (t2 + 1.0)
            den = jnp.where(t2 < 0.5, 1.0, (t2 + 1.0) * t2)
            gneg = jnp.where(t2 < 0.5, 1.0, (pf - cvec) / den)
            contrib = relu * jnp.where(lab > 0, gpos, gneg)
            return (c_run + jnp.sum(lab), lacc + contrib)
        _, lacc = lax.fori_loop(0, SUB, l2,
                                (jnp.int32(0), jnp.zeros((16,), jnp.float32)))
        part = jnp.sum(lacc)
        stgf[...] = jnp.full((16,), part, jnp.float32)
        pltpu.sync_copy(stgf, gRed.at[s])
        plsc.subcore_barrier()

        @pl.when(s == 0)
        def _():
            pltpu.sync_copy(gRed, gredv)
            pv = plsc.load_gather(gredv, [iota, zer16])
            stgf[...] = jnp.full((16,), jnp.sum(pv), jnp.float32)
            pltpu.sync_copy(stgf, out_hbm.at[p])
        plsc.subcore_barrier()
        return carry_outer

    lax.fori_loop(0, NPAIR // 2, do_pair, 0)


def kernel(d0, d1, d2, d3, d4, r_glass, r_global, ground_truth):
    dflat = [x.reshape(B, H, W) for x in (d0, d1, d2, d3, d4)]
    gt2 = ground_truth.reshape(B, H, W)
    rgl = r_glass.reshape(B, 3, H, W)
    rgo = r_global.reshape(B, 3, H, W)

    ROWS = H // NC_TC
    lin = pl.BlockSpec((1, ROWS, W), lambda i, j: (i, j, 0))
    rin = pl.BlockSpec((1, 3, ROWS, W), lambda i, j: (i, 0, j, 0))
    kall, msep = pl.pallas_call(
        _prep_body,
        grid=(B, NC_TC),
        in_specs=[lin] * 5 + [lin, rin, rin],
        out_specs=[pl.BlockSpec((5, 1, ROWS, W), lambda i, j: (0, i, j, 0)),
                   pl.BlockSpec((1, 1, 1), lambda i, j: (i * NC_TC + j, 0, 0),
                                memory_space=pltpu.SMEM)],
        out_shape=[jax.ShapeDtypeStruct((5, B, H, W), jnp.int32),
                   jax.ShapeDtypeStruct((B * NC_TC, 1, 1), jnp.float32)],
    )(*dflat, gt2, rgl, rgo)

    keys_flat = kall.reshape(NPAIR * N)

    mesh = plsc.VectorSubcoreMesh(core_axis_name="c", subcore_axis_name="s")
    sc_fn = pl.kernel(
        _sc_body,
        out_type=jax.ShapeDtypeStruct((NPAIR, 16), jnp.float32),
        mesh=mesh,
        scratch_types=[
            pltpu.VMEM_SHARED((N,), jnp.int32),
            pltpu.VMEM_SHARED((N,), jnp.int32),
            pltpu.VMEM_SHARED((16, RAD), jnp.int32),
            pltpu.VMEM_SHARED((16, 16), jnp.int32),
            pltpu.VMEM_SHARED((16, 16), jnp.float32),
            pltpu.VMEM((CHUNK,), jnp.int32),
            pltpu.VMEM((RAD * 16,), jnp.int32),
            pltpu.VMEM((RAD * 16,), jnp.int32),
            pltpu.VMEM((16, RAD), jnp.int32),
            pltpu.VMEM((16, 16), jnp.int32),
            pltpu.VMEM((16, 16), jnp.float32),
            pltpu.VMEM((CHUNK,), jnp.int32),
            pltpu.VMEM((RAD,), jnp.int32),
            pltpu.VMEM((16,), jnp.int32),
            pltpu.VMEM((16,), jnp.float32),
        ],
        compiler_params=pltpu.CompilerParams(needs_layout_passes=False),
    )
    lov = sc_fn(keys_flat)

    mse = jnp.sum(msep) / (B * 3 * H * W)
    return mse + jnp.sum(lov[:, 0]) / B


# speedup vs baseline: 4.2470x; 1.0831x over previous
import jax
import jax.numpy as jnp
from jax import lax
from jax.experimental import pallas as pl
from jax.experimental.pallas import tpu as pltpu
from jax.experimental.pallas import tpu_sc as plsc

B, H, W = 4, 512, 512
N = H * W
NPAIR = 20
NTILE = 16
CHUNK = N // NTILE
SUB = CHUNK // 16
RAD = 256
NC_TC = 8
C_TC = N // NC_TC
import numpy as np
MIN32 = np.int32(-2147483648)


def _prep_body(d0, d1, d2, d3, d4, gt, rgl, rgo, kall, msep):
    gti = gt[...]
    gtf = gti.astype(jnp.float32)
    sign = 2.0 * gtf - 1.0
    for dd, dref in enumerate((d0, d1, d2, d3, d4)):
        x = dref[...]
        e = 1.0 - x * sign
        bi = lax.bitcast_convert_type(e, jnp.int32)
        srt = jnp.where(bi < 0, jnp.bitwise_xor(bi, -1),
                        jnp.bitwise_xor(bi, MIN32))
        k = jnp.bitwise_xor(srt, -1)
        k = jnp.bitwise_or(jnp.bitwise_and(k, -2), gti)
        kall[dd] = k
    g3 = gtf[:, None]
    df = (rgo[...] - rgl[...]) * g3
    msep[0, 0, 0] = jnp.sum(df * df)


def _sc_body(keys_hbm, out_hbm, bufA, bufB, gT, gLab, gRed,
             tk, hist, bases, gtl, glabv, gredv, posb, stg, stgi, stgf):
    c = lax.axis_index("c")
    s = lax.axis_index("s")
    iota = lax.iota(jnp.int32, 16)
    ones = jnp.ones((16,), jnp.int32)
    zer16 = jnp.zeros((16,), jnp.int32)

    def do_pair(jp, carry_outer):
        p = 2 * jp + c
        base_off = p * N

        for pass_i in range(4):
            if pass_i == 0:
                off = pl.multiple_of(base_off + s * CHUNK, CHUNK)
                pltpu.sync_copy(keys_hbm.at[pl.ds(off, CHUNK)], tk)
                dst = bufB
            else:
                src = bufB if pass_i % 2 == 1 else bufA
                dst = bufA if pass_i % 2 == 1 else bufB
                off = pl.multiple_of(s * CHUNK, CHUNK)
                pltpu.sync_copy(src.at[pl.ds(off, CHUNK)], tk)
            shift = jnp.full((16,), 8 * pass_i, jnp.int32)

            def zb(i, _):
                plsc.store_scatter(hist, [i * 16 + iota], zer16)
                return 0
            lax.fori_loop(0, RAD, zb, 0)

            def hb(jj, _):
                idxv = iota * SUB + jj
                kv = plsc.load_gather(tk, [idxv])
                dg = jnp.bitwise_and(lax.shift_right_logical(kv, shift), 255)
                plsc.addupdate_scatter(hist, [dg * 16 + iota], ones)
                return 0
            lax.fori_loop(0, SUB, hb, 0, unroll=4)

            def mg(g, _):
                dd = g * 16 + iota
                acc = zer16
                for l in range(16):
                    acc = acc + plsc.load_gather(hist, [dd * 16 + l])
                plsc.store_scatter(stg, [g * 16 + iota], acc)
                return 0
            lax.fori_loop(0, 16, mg, 0)
            pltpu.sync_copy(stg, gT.at[s])
            plsc.subcore_barrier()
            pltpu.sync_copy(gT, gtl)

            def sb(d, carry):
                dvec = jnp.full((16,), d, jnp.int32)
                tvec = plsc.load_gather(gtl, [iota, dvec])
                tinc = plsc.cumsum(tvec)
                tex_t = jnp.sum(jnp.where(iota == s, tinc - tvec, 0))
                hvec = plsc.load_gather(hist, [d * 16 + iota])
                hex_ = plsc.cumsum(hvec) - hvec
                plsc.store_scatter(bases, [d * 16 + iota],
                                   carry + tex_t + hex_)
                return carry + jnp.sum(tvec)
            lax.fori_loop(0, RAD, sb, jnp.int32(0))

            def pb(jj, _):
                idxv = iota * SUB + jj
                kv = plsc.load_gather(tk, [idxv])
                dg = jnp.bitwise_and(lax.shift_right_logical(kv, shift), 255)
                fl = dg * 16 + iota
                pos = plsc.load_gather(bases, [fl])
                plsc.store_scatter(bases, [fl], pos + ones)
                plsc.store_scatter(posb, [idxv], pos)
                return 0
            lax.fori_loop(0, SUB, pb, 0, unroll=4)
            pltpu.sync_copy(tk, dst.at[posb])
            plsc.subcore_barrier()

        off = pl.multiple_of(s * CHUNK, CHUNK)
        pltpu.sync_copy(bufA.at[pl.ds(off, CHUNK)], tk)

        def l1(jj, acc):
            kv = plsc.load_gather(tk, [jj * 16 + iota])
            return acc + jnp.bitwise_and(kv, 1)
        accv = lax.fori_loop(0, SUB, l1, zer16, unroll=8)
        tot = jnp.sum(accv)
        stgi[...] = jnp.full((16,), tot, jnp.int32)
        pltpu.sync_copy(stgi, gLab.at[s])
        plsc.subcore_barrier()
        pltpu.sync_copy(gLab, glabv)
        totv = plsc.load_gather(glabv, [iota, zer16])
        pcount = jnp.sum(totv)
        cbase = jnp.sum(jnp.where(iota == s, plsc.cumsum(totv) - totv, 0))
        pf = pcount.astype(jnp.float32)

        def l2(jj, carry):
            c_run, lacc = carry
            kv = plsc.load_gather(tk, [jj * 16 + iota])
            lab = jnp.bitwise_and(kv, 1)
            s2 = jnp.bitwise_xor(kv, -1)
            u2 = jnp.where(s2 < 0, jnp.bitwise_xor(s2, MIN32),
                           jnp.bitwise_xor(s2, -1))
            e = plsc.bitcast(u2, jnp.float32)
            relu = jnp.maximum(e, 0.0)
            cvec = (c_run + cbase + plsc.cumsum(lab)).astype(jnp.float32)
            ivec = (s * CHUNK + jj * 16 + iota).astype(jnp.float32)
            t2 = ivec + pf - cvec
            ispos = lab > 0
            bad = jnp.logical_and(t2 < 0.5, jnp.logical_not(ispos))
            num = jnp.where(ispos, 1.0, pf - cvec)
            num = jnp.where(bad, 1.0, num)
            den = jnp.where(ispos, t2 + 1.0, (t2 + 1.0) * t2)
            den = jnp.where(bad, 1.0, den)
            contrib = relu * num / den
            return (c_run + jnp.sum(lab), lacc + contrib)
        _, lacc = lax.fori_loop(0, SUB, l2,
                                (jnp.int32(0), jnp.zeros((16,), jnp.float32)),
                                unroll=4)
        part = jnp.sum(lacc)
        stgf[...] = jnp.full((16,), part, jnp.float32)
        pltpu.sync_copy(stgf, gRed.at[s])
        plsc.subcore_barrier()

        @pl.when(s == 0)
        def _():
            pltpu.sync_copy(gRed, gredv)
            pv = plsc.load_gather(gredv, [iota, zer16])
            stgf[...] = jnp.full((16,), jnp.sum(pv), jnp.float32)
            pltpu.sync_copy(stgf, out_hbm.at[p])
        plsc.subcore_barrier()
        return carry_outer

    lax.fori_loop(0, NPAIR // 2, do_pair, 0)


def kernel(d0, d1, d2, d3, d4, r_glass, r_global, ground_truth):
    dflat = [x.reshape(B, H, W) for x in (d0, d1, d2, d3, d4)]
    gt2 = ground_truth.reshape(B, H, W)
    rgl = r_glass.reshape(B, 3, H, W)
    rgo = r_global.reshape(B, 3, H, W)

    ROWS = H // NC_TC
    lin = pl.BlockSpec((1, ROWS, W), lambda i, j: (i, j, 0))
    rin = pl.BlockSpec((1, 3, ROWS, W), lambda i, j: (i, 0, j, 0))
    kall, msep = pl.pallas_call(
        _prep_body,
        grid=(B, NC_TC),
        in_specs=[lin] * 5 + [lin, rin, rin],
        out_specs=[pl.BlockSpec((5, 1, ROWS, W), lambda i, j: (0, i, j, 0)),
                   pl.BlockSpec((1, 1, 1), lambda i, j: (i * NC_TC + j, 0, 0),
                                memory_space=pltpu.SMEM)],
        out_shape=[jax.ShapeDtypeStruct((5, B, H, W), jnp.int32),
                   jax.ShapeDtypeStruct((B * NC_TC, 1, 1), jnp.float32)],
    )(*dflat, gt2, rgl, rgo)

    keys_flat = kall.reshape(NPAIR * N)

    mesh = plsc.VectorSubcoreMesh(core_axis_name="c", subcore_axis_name="s")
    sc_fn = pl.kernel(
        _sc_body,
        out_type=jax.ShapeDtypeStruct((NPAIR, 16), jnp.float32),
        mesh=mesh,
        scratch_types=[
            pltpu.VMEM_SHARED((N,), jnp.int32),
            pltpu.VMEM_SHARED((N,), jnp.int32),
            pltpu.VMEM_SHARED((16, RAD), jnp.int32),
            pltpu.VMEM_SHARED((16, 16), jnp.int32),
            pltpu.VMEM_SHARED((16, 16), jnp.float32),
            pltpu.VMEM((CHUNK,), jnp.int32),
            pltpu.VMEM((RAD * 16,), jnp.int32),
            pltpu.VMEM((RAD * 16,), jnp.int32),
            pltpu.VMEM((16, RAD), jnp.int32),
            pltpu.VMEM((16, 16), jnp.int32),
            pltpu.VMEM((16, 16), jnp.float32),
            pltpu.VMEM((CHUNK,), jnp.int32),
            pltpu.VMEM((RAD,), jnp.int32),
            pltpu.VMEM((16,), jnp.int32),
            pltpu.VMEM((16,), jnp.float32),
        ],
        compiler_params=pltpu.CompilerParams(needs_layout_passes=False),
    )
    lov = sc_fn(keys_flat)

    mse = jnp.sum(msep) / (B * 3 * H * W)
    return mse + jnp.sum(lov[:, 0]) / B


# speedup vs baseline: 5.3289x; 1.2547x over previous
import jax
import jax.numpy as jnp
import numpy as np
from jax import lax
from jax.experimental import pallas as pl
from jax.experimental.pallas import tpu as pltpu
from jax.experimental.pallas import tpu_sc as plsc

B, H, W = 4, 512, 512
N = H * W
NPAIR = 20
NTILE = 16
CHUNK = N // NTILE
SUB = CHUNK // 16
NBANK = 4
QSUB = SUB // NBANK
QTILE = CHUNK // NBANK
RAD = 256
NC_TC = 8
MIN32 = np.int32(-2147483648)


def _prep_body(d0, d1, d2, d3, d4, gt, rgl, rgo, kall, msep):
    gti = gt[...]
    gtf = gti.astype(jnp.float32)
    sign = 2.0 * gtf - 1.0
    for dd, dref in enumerate((d0, d1, d2, d3, d4)):
        x = dref[...]
        e = 1.0 - x * sign
        bi = lax.bitcast_convert_type(e, jnp.int32)
        srt = jnp.where(bi < 0, jnp.bitwise_xor(bi, -1),
                        jnp.bitwise_xor(bi, MIN32))
        k = jnp.bitwise_xor(srt, -1)
        k = jnp.bitwise_or(jnp.bitwise_and(k, -2), gti)
        kall[dd] = k
    g3 = gtf[:, None]
    df = (rgo[...] - rgl[...]) * g3
    msep[0, 0, 0] = jnp.sum(df * df)


_GDN = lax.GatherDimensionNumbers(offset_dims=(), collapsed_slice_dims=(0,),
                                  start_index_map=(0,))


def _lane_bcast(v, idx):
    return lax.gather(v, idx[:, None], dimension_numbers=_GDN,
                      slice_sizes=(1,),
                      mode=lax.GatherScatterMode.PROMISE_IN_BOUNDS)


def _sc_body(keys_hbm, out_hbm, bufA, bufB, gT, gLab, gRed,
             tk, h0, h1, h2, h3, b0, b1, b2, b3, g0, g1, g2, g3,
             gtl, glabv, gredv, posb, stg, stgi, stgf):
    c = lax.axis_index("c")
    s = lax.axis_index("s")
    iota = lax.iota(jnp.int32, 16)
    iota_sub = iota * SUB
    iota_rad = iota * RAD
    ones = jnp.ones((16,), jnp.int32)
    zer16 = jnp.zeros((16,), jnp.int32)
    hists = (h0, h1, h2, h3)
    bass = (b0, b1, b2, b3)
    dgbs = (g0, g1, g2, g3)

    def z0(i, _):
        for hk in hists:
            plsc.store_scatter(hk, [i * 16 + iota], zer16)
        return 0
    lax.fori_loop(0, RAD * 16 // 16, z0, 0)

    def do_pair(jp, carry_outer):
        p = 2 * jp + c
        base_off = p * N

        for pass_i in range(4):
            if pass_i == 0:
                off = pl.multiple_of(base_off + s * CHUNK, CHUNK)
                pltpu.sync_copy(keys_hbm.at[pl.ds(off, CHUNK)], tk)
                dst = bufB
            else:
                src = bufB if pass_i % 2 == 1 else bufA
                dst = bufA if pass_i % 2 == 1 else bufB
                off = pl.multiple_of(s * CHUNK, CHUNK)
                pltpu.sync_copy(src.at[pl.ds(off, CHUNK)], tk)
            shift = jnp.full((16,), 8 * pass_i, jnp.int32)

            def hb(jj, _):
                for k in range(NBANK):
                    idxv = iota_sub + (k * QSUB) + jj
                    kv = plsc.load_gather(tk, [idxv])
                    dg = jnp.bitwise_and(
                        lax.shift_right_logical(kv, shift), 255)
                    fl = iota_rad + dg
                    plsc.addupdate_scatter(hists[k], [fl], ones)
                    plsc.store_scatter(dgbs[k], [jj * 16 + iota], fl)
                return 0
            lax.fori_loop(0, QSUB, hb, 0, unroll=2)

            def mg(g, _):
                gbase = g * 16 + iota
                tot = zer16
                for l in range(16):
                    for k in range(NBANK):
                        tot = tot + plsc.load_gather(
                            hists[k], [l * RAD + gbase])
                plsc.store_scatter(stg, [gbase], tot)
                return 0
            lax.fori_loop(0, 16, mg, 0)
            pltpu.sync_copy(stg, gT.at[s])
            plsc.subcore_barrier()
            pltpu.sync_copy(gT, gtl)

            def sb(g, carry):
                gbase = g * 16 + iota
                tots = zer16
                pre = zer16
                for t in range(16):
                    row = plsc.load_gather(
                        gtl, [jnp.full((16,), t, jnp.int32), gbase])
                    tots = tots + row
                    pre = pre + jnp.where(t < s, row, 0)
                ex = plsc.cumsum(tots) - tots
                base_dig = carry + pre + ex
                hacc = zer16
                for l in range(16):
                    for k in range(NBANK):
                        hseg = plsc.load_gather(hists[k], [l * RAD + gbase])
                        plsc.store_scatter(bass[k], [l * RAD + gbase],
                                           base_dig + hacc)
                        hacc = hacc + hseg
                        plsc.store_scatter(hists[k], [l * RAD + gbase],
                                           zer16)
                return carry + jnp.sum(tots)
            lax.fori_loop(0, 16, sb, jnp.int32(0))

            def pb(jj, _):
                for k in range(NBANK):
                    fl = plsc.load_gather(dgbs[k], [jj * 16 + iota])
                    pos = plsc.load_gather(bass[k], [fl])
                    plsc.store_scatter(bass[k], [fl], pos + ones)
                    plsc.store_scatter(posb, [iota_sub + (k * QSUB) + jj],
                                       pos)
                return 0
            lax.fori_loop(0, QSUB, pb, 0, unroll=2)
            pltpu.sync_copy(tk, dst.at[posb])
            plsc.subcore_barrier()

        off = pl.multiple_of(s * CHUNK, CHUNK)
        pltpu.sync_copy(bufA.at[pl.ds(off, CHUNK)], tk)

        def l1(jj, accs):
            out = []
            for k in range(NBANK):
                kv = plsc.load_gather(tk, [k * QTILE + jj * 16 + iota])
                out.append(accs[k] + jnp.bitwise_and(kv, 1))
            return tuple(out)
        accs = lax.fori_loop(0, QTILE // 16, l1,
                             (zer16,) * NBANK, unroll=2)
        q = [jnp.sum(a) for a in accs]
        tot = q[0] + q[1] + q[2] + q[3]
        stgi[...] = jnp.full((16,), tot, jnp.int32)
        pltpu.sync_copy(stgi, gLab.at[s])
        plsc.subcore_barrier()
        pltpu.sync_copy(gLab, glabv)
        totv = plsc.load_gather(glabv, [iota, zer16])
        pcount = jnp.sum(totv)
        cbase = jnp.sum(jnp.where(iota == s, plsc.cumsum(totv) - totv, 0))
        pf = pcount.astype(jnp.float32)
        cb = [cbase, cbase + q[0], cbase + q[0] + q[1],
              cbase + q[0] + q[1] + q[2]]

        ftn = jnp.full((16,), 15, jnp.int32)

        def l2(jj, carry):
            cruns, laccs = carry
            ncr, nla = [], []
            for k in range(NBANK):
                kv = plsc.load_gather(tk, [k * QTILE + jj * 16 + iota])
                lab = jnp.bitwise_and(kv, 1)
                s2 = jnp.bitwise_xor(kv, -1)
                u2 = jnp.where(s2 < 0, jnp.bitwise_xor(s2, MIN32),
                               jnp.bitwise_xor(s2, -1))
                e = plsc.bitcast(u2, jnp.float32)
                relu = jnp.maximum(e, 0.0)
                inc = plsc.cumsum(lab)
                cvec = (cruns[k] + cb[k] + inc).astype(jnp.float32)
                ivec = (s * CHUNK + k * QTILE + jj * 16 + iota
                        ).astype(jnp.float32)
                t2 = ivec + pf - cvec
                ispos = lab > 0
                bad = jnp.logical_and(t2 < 0.5, jnp.logical_not(ispos))
                numr = jnp.where(ispos, 1.0, pf - cvec)
                numr = jnp.where(bad, 1.0, numr)
                den = jnp.where(ispos, t2 + 1.0, (t2 + 1.0) * t2)
                den = jnp.where(bad, 1.0, den)
                nla.append(laccs[k] + relu * numr / den)
                ncr.append(cruns[k] + _lane_bcast(inc, ftn))
            return (tuple(ncr), tuple(nla))

        zf = jnp.zeros((16,), jnp.float32)
        _, laccs = lax.fori_loop(0, QTILE // 16, l2,
                                 ((zer16,) * NBANK, (zf,) * NBANK),
                                 unroll=2)
        part = jnp.sum(laccs[0] + laccs[1] + laccs[2] + laccs[3])
        stgf[...] = jnp.full((16,), part, jnp.float32)
        pltpu.sync_copy(stgf, gRed.at[s])
        plsc.subcore_barrier()

        @pl.when(s == 0)
        def _():
            pltpu.sync_copy(gRed, gredv)
            pv = plsc.load_gather(gredv, [iota, zer16])
            stgf[...] = jnp.full((16,), jnp.sum(pv), jnp.float32)
            pltpu.sync_copy(stgf, out_hbm.at[p])
        plsc.subcore_barrier()
        return carry_outer

    lax.fori_loop(0, NPAIR // 2, do_pair, 0)


def kernel(d0, d1, d2, d3, d4, r_glass, r_global, ground_truth):
    dflat = [x.reshape(B, H, W) for x in (d0, d1, d2, d3, d4)]
    gt2 = ground_truth.reshape(B, H, W)
    rgl = r_glass.reshape(B, 3, H, W)
    rgo = r_global.reshape(B, 3, H, W)

    ROWS = H // NC_TC
    lin = pl.BlockSpec((1, ROWS, W), lambda i, j: (i, j, 0))
    rin = pl.BlockSpec((1, 3, ROWS, W), lambda i, j: (i, 0, j, 0))
    kall, msep = pl.pallas_call(
        _prep_body,
        grid=(B, NC_TC),
        in_specs=[lin] * 5 + [lin, rin, rin],
        out_specs=[pl.BlockSpec((5, 1, ROWS, W), lambda i, j: (0, i, j, 0)),
                   pl.BlockSpec((1, 1, 1), lambda i, j: (i * NC_TC + j, 0, 0),
                                memory_space=pltpu.SMEM)],
        out_shape=[jax.ShapeDtypeStruct((5, B, H, W), jnp.int32),
                   jax.ShapeDtypeStruct((B * NC_TC, 1, 1), jnp.float32)],
    )(*dflat, gt2, rgl, rgo)

    keys_flat = kall.reshape(NPAIR * N)

    mesh = plsc.VectorSubcoreMesh(core_axis_name="c", subcore_axis_name="s")
    sc_fn = pl.kernel(
        _sc_body,
        out_type=jax.ShapeDtypeStruct((NPAIR, 16), jnp.float32),
        mesh=mesh,
        scratch_types=[
            pltpu.VMEM_SHARED((N,), jnp.int32),
            pltpu.VMEM_SHARED((N,), jnp.int32),
            pltpu.VMEM_SHARED((16, RAD), jnp.int32),
            pltpu.VMEM_SHARED((16, 16), jnp.int32),
            pltpu.VMEM_SHARED((16, 16), jnp.float32),
            pltpu.VMEM((CHUNK,), jnp.int32),
            pltpu.VMEM((RAD * 16,), jnp.int32),
            pltpu.VMEM((RAD * 16,), jnp.int32),
            pltpu.VMEM((RAD * 16,), jnp.int32),
            pltpu.VMEM((RAD * 16,), jnp.int32),
            pltpu.VMEM((RAD * 16,), jnp.int32),
            pltpu.VMEM((RAD * 16,), jnp.int32),
            pltpu.VMEM((RAD * 16,), jnp.int32),
            pltpu.VMEM((RAD * 16,), jnp.int32),
            pltpu.VMEM((QSUB * 16,), jnp.int32),
            pltpu.VMEM((QSUB * 16,), jnp.int32),
            pltpu.VMEM((QSUB * 16,), jnp.int32),
            pltpu.VMEM((QSUB * 16,), jnp.int32),
            pltpu.VMEM((16, RAD), jnp.int32),
            pltpu.VMEM((16, 16), jnp.int32),
            pltpu.VMEM((16, 16), jnp.float32),
            pltpu.VMEM((CHUNK,), jnp.int32),
            pltpu.VMEM((RAD,), jnp.int32),
            pltpu.VMEM((16,), jnp.int32),
            pltpu.VMEM((16,), jnp.float32),
        ],
        compiler_params=pltpu.CompilerParams(needs_layout_passes=False),
    )
    lov = sc_fn(keys_flat)

    mse = jnp.sum(msep) / (B * 3 * H * W)
    return mse + jnp.sum(lov[:, 0]) / B


# speedup vs baseline: 6.6500x; 1.2479x over previous
import jax
import jax.numpy as jnp
import numpy as np
from jax import lax
from jax.experimental import pallas as pl
from jax.experimental.pallas import tpu as pltpu
from jax.experimental.pallas import tpu_sc as plsc

B, H, W = 4, 512, 512
N = H * W
NPAIR = 20
NTILE = 16
CHUNK = N // NTILE
SUB = CHUNK // 16
NBANK = 4
QSUB = SUB // NBANK
QTILE = CHUNK // NBANK
RAD = 256
NC_TC = 8
MIN32 = np.int32(-2147483648)


def _prep_body(d0, d1, d2, d3, d4, gt, rgl, rgo, kall, msep):
    gti = gt[...]
    gtf = gti.astype(jnp.float32)
    sign = 2.0 * gtf - 1.0
    for dd, dref in enumerate((d0, d1, d2, d3, d4)):
        x = dref[...]
        e = 1.0 - x * sign
        bi = lax.bitcast_convert_type(e, jnp.int32)
        srt = jnp.where(bi < 0, jnp.bitwise_xor(bi, -1),
                        jnp.bitwise_xor(bi, MIN32))
        k = jnp.bitwise_xor(srt, -1)
        k = jnp.bitwise_or(jnp.bitwise_and(k, -2), gti)
        kall[dd] = k
    g3 = gtf[:, None]
    df = (rgo[...] - rgl[...]) * g3
    msep[0, 0, 0] = jnp.sum(df * df)


_GDN = lax.GatherDimensionNumbers(offset_dims=(), collapsed_slice_dims=(0,),
                                  start_index_map=(0,))


def _lane_bcast(v, idx):
    return lax.gather(v, idx[:, None], dimension_numbers=_GDN,
                      slice_sizes=(1,),
                      mode=lax.GatherScatterMode.PROMISE_IN_BOUNDS)


def _sc_body(keys_hbm, out_hbm, bufA, bufB, gT, gLab, gRed,
             tk, h0, h1, h2, h3, b0, b1, b2, b3, g0, g1, g2, g3,
             gtl, glabv, gredv, posb, stg, stgi, stgf):
    c = lax.axis_index("c")
    s = lax.axis_index("s")
    iota = lax.iota(jnp.int32, 16)
    iota_sub = iota * SUB
    iota_rad = iota * RAD
    ones = jnp.ones((16,), jnp.int32)
    zer16 = jnp.zeros((16,), jnp.int32)
    hists = (h0, h1, h2, h3)
    bass = (b0, b1, b2, b3)
    dgbs = (g0, g1, g2, g3)

    def z0(i, _):
        for hk in hists:
            plsc.store_scatter(hk, [i * 16 + iota], zer16)
        return 0
    lax.fori_loop(0, RAD * 16 // 16, z0, 0)

    def do_pair(jp, carry_outer):
        p = 2 * jp + c
        base_off = p * N

        for pass_i in range(4):
            if pass_i == 0:
                off = pl.multiple_of(base_off + s * CHUNK, CHUNK)
                pltpu.sync_copy(keys_hbm.at[pl.ds(off, CHUNK)], tk)
                dst = bufB
            else:
                src = bufB if pass_i % 2 == 1 else bufA
                dst = bufA if pass_i % 2 == 1 else bufB
                off = pl.multiple_of(s * CHUNK, CHUNK)
                pltpu.sync_copy(src.at[pl.ds(off, CHUNK)], tk)
            shift = jnp.full((16,), 8 * pass_i, jnp.int32)

            @plsc.parallel_loop(0, QSUB, unroll=4)
            def _(jj):
                for k in range(NBANK):
                    idxv = iota_sub + (k * QSUB) + jj
                    kv = plsc.load_gather(tk, [idxv])
                    dg = jnp.bitwise_and(
                        lax.shift_right_logical(kv, shift), 255)
                    fl = iota_rad + dg
                    plsc.addupdate_scatter(hists[k], [fl], ones)
                    plsc.store_scatter(dgbs[k], [jj * 16 + iota], fl)

            @plsc.parallel_loop(0, 16, unroll=2)
            def _(g):
                gbase = g * 16 + iota
                tot = zer16
                for l in range(16):
                    for k in range(NBANK):
                        tot = tot + plsc.load_gather(
                            hists[k], [l * RAD + gbase])
                plsc.store_scatter(stg, [gbase], tot)
            pltpu.sync_copy(stg, gT.at[s])
            plsc.subcore_barrier()
            pltpu.sync_copy(gT, gtl)

            def sb(g, carry):
                gbase = g * 16 + iota
                tots = zer16
                pre = zer16
                for t in range(16):
                    row = plsc.load_gather(
                        gtl, [jnp.full((16,), t, jnp.int32), gbase])
                    tots = tots + row
                    pre = pre + jnp.where(t < s, row, 0)
                ex = plsc.cumsum(tots) - tots
                base_dig = carry + pre + ex
                hacc = zer16
                for l in range(16):
                    for k in range(NBANK):
                        hseg = plsc.load_gather(hists[k], [l * RAD + gbase])
                        plsc.store_scatter(bass[k], [l * RAD + gbase],
                                           base_dig + hacc)
                        hacc = hacc + hseg
                        plsc.store_scatter(hists[k], [l * RAD + gbase],
                                           zer16)
                return carry + jnp.sum(tots)
            lax.fori_loop(0, 16, sb, jnp.int32(0))

            def pb(jj, _):
                for k in range(NBANK):
                    fl = plsc.load_gather(dgbs[k], [jj * 16 + iota])
                    pos = plsc.load_gather(bass[k], [fl])
                    plsc.store_scatter(bass[k], [fl], pos + ones)
                    plsc.store_scatter(posb, [iota_sub + (k * QSUB) + jj],
                                       pos)
                return 0
            lax.fori_loop(0, QSUB, pb, 0, unroll=4)
            pltpu.sync_copy(tk, dst.at[posb])
            plsc.subcore_barrier()

        off = pl.multiple_of(s * CHUNK, CHUNK)
        pltpu.sync_copy(bufA.at[pl.ds(off, CHUNK)], tk)

        def l1(jj, accs):
            out = []
            for k in range(NBANK):
                kv = plsc.load_gather(tk, [k * QTILE + jj * 16 + iota])
                out.append(accs[k] + jnp.bitwise_and(kv, 1))
            return tuple(out)
        accs = lax.fori_loop(0, QTILE // 16, l1,
                             (zer16,) * NBANK, unroll=2)
        q = [jnp.sum(a) for a in accs]
        tot = q[0] + q[1] + q[2] + q[3]
        stgi[...] = jnp.full((16,), tot, jnp.int32)
        pltpu.sync_copy(stgi, gLab.at[s])
        plsc.subcore_barrier()
        pltpu.sync_copy(gLab, glabv)
        totv = plsc.load_gather(glabv, [iota, zer16])
        pcount = jnp.sum(totv)
        cbase = jnp.sum(jnp.where(iota == s, plsc.cumsum(totv) - totv, 0))
        pf = pcount.astype(jnp.float32)
        cb = [cbase, cbase + q[0], cbase + q[0] + q[1],
              cbase + q[0] + q[1] + q[2]]

        ftn = jnp.full((16,), 15, jnp.int32)

        def l2(jj, carry):
            cruns, laccs = carry
            ncr, nla = [], []
            for k in range(NBANK):
                kv = plsc.load_gather(tk, [k * QTILE + jj * 16 + iota])
                lab = jnp.bitwise_and(kv, 1)
                s2 = jnp.bitwise_xor(kv, -1)
                u2 = jnp.where(s2 < 0, jnp.bitwise_xor(s2, MIN32),
                               jnp.bitwise_xor(s2, -1))
                e = plsc.bitcast(u2, jnp.float32)
                relu = jnp.maximum(e, 0.0)
                inc = plsc.cumsum(lab)
                cvec = (cruns[k] + cb[k] + inc).astype(jnp.float32)
                ivec = (s * CHUNK + k * QTILE + jj * 16 + iota
                        ).astype(jnp.float32)
                t2 = ivec + pf - cvec
                ispos = lab > 0
                bad = jnp.logical_and(t2 < 0.5, jnp.logical_not(ispos))
                numr = jnp.where(ispos, 1.0, pf - cvec)
                numr = jnp.where(bad, 1.0, numr)
                den = jnp.where(ispos, t2 + 1.0, (t2 + 1.0) * t2)
                den = jnp.where(bad, 1.0, den)
                nla.append(laccs[k] + relu * numr / den)
                ncr.append(cruns[k] + _lane_bcast(inc, ftn))
            return (tuple(ncr), tuple(nla))

        zf = jnp.zeros((16,), jnp.float32)
        _, laccs = lax.fori_loop(0, QTILE // 16, l2,
                                 ((zer16,) * NBANK, (zf,) * NBANK),
                                 unroll=2)
        part = jnp.sum(laccs[0] + laccs[1] + laccs[2] + laccs[3])
        stgf[...] = jnp.full((16,), part, jnp.float32)
        pltpu.sync_copy(stgf, gRed.at[s])
        plsc.subcore_barrier()

        @pl.when(s == 0)
        def _():
            pltpu.sync_copy(gRed, gredv)
            pv = plsc.load_gather(gredv, [iota, zer16])
            stgf[...] = jnp.full((16,), jnp.sum(pv), jnp.float32)
            pltpu.sync_copy(stgf, out_hbm.at[p])
        plsc.subcore_barrier()
        return carry_outer

    lax.fori_loop(0, NPAIR // 2, do_pair, 0)


def kernel(d0, d1, d2, d3, d4, r_glass, r_global, ground_truth):
    dflat = [x.reshape(B, H, W) for x in (d0, d1, d2, d3, d4)]
    gt2 = ground_truth.reshape(B, H, W)
    rgl = r_glass.reshape(B, 3, H, W)
    rgo = r_global.reshape(B, 3, H, W)

    ROWS = H // NC_TC
    lin = pl.BlockSpec((1, ROWS, W), lambda i, j: (i, j, 0))
    rin = pl.BlockSpec((1, 3, ROWS, W), lambda i, j: (i, 0, j, 0))
    kall, msep = pl.pallas_call(
        _prep_body,
        grid=(B, NC_TC),
        in_specs=[lin] * 5 + [lin, rin, rin],
        out_specs=[pl.BlockSpec((5, 1, ROWS, W), lambda i, j: (0, i, j, 0)),
                   pl.BlockSpec((1, 1, 1), lambda i, j: (i * NC_TC + j, 0, 0),
                                memory_space=pltpu.SMEM)],
        out_shape=[jax.ShapeDtypeStruct((5, B, H, W), jnp.int32),
                   jax.ShapeDtypeStruct((B * NC_TC, 1, 1), jnp.float32)],
    )(*dflat, gt2, rgl, rgo)

    keys_flat = kall.reshape(NPAIR * N)

    mesh = plsc.VectorSubcoreMesh(core_axis_name="c", subcore_axis_name="s")
    sc_fn = pl.kernel(
        _sc_body,
        out_type=jax.ShapeDtypeStruct((NPAIR, 16), jnp.float32),
        mesh=mesh,
        scratch_types=[
            pltpu.VMEM_SHARED((N,), jnp.int32),
            pltpu.VMEM_SHARED((N,), jnp.int32),
            pltpu.VMEM_SHARED((16, RAD), jnp.int32),
            pltpu.VMEM_SHARED((16, 16), jnp.int32),
            pltpu.VMEM_SHARED((16, 16), jnp.float32),
            pltpu.VMEM((CHUNK,), jnp.int32),
            pltpu.VMEM((RAD * 16,), jnp.int32),
            pltpu.VMEM((RAD * 16,), jnp.int32),
            pltpu.VMEM((RAD * 16,), jnp.int32),
            pltpu.VMEM((RAD * 16,), jnp.int32),
            pltpu.VMEM((RAD * 16,), jnp.int32),
            pltpu.VMEM((RAD * 16,), jnp.int32),
            pltpu.VMEM((RAD * 16,), jnp.int32),
            pltpu.VMEM((RAD * 16,), jnp.int32),
            pltpu.VMEM((QSUB * 16,), jnp.int32),
            pltpu.VMEM((QSUB * 16,), jnp.int32),
            pltpu.VMEM((QSUB * 16,), jnp.int32),
            pltpu.VMEM((QSUB * 16,), jnp.int32),
            pltpu.VMEM((16, RAD), jnp.int32),
            pltpu.VMEM((16, 16), jnp.int32),
            pltpu.VMEM((16, 16), jnp.float32),
            pltpu.VMEM((CHUNK,), jnp.int32),
            pltpu.VMEM((RAD,), jnp.int32),
            pltpu.VMEM((16,), jnp.int32),
            pltpu.VMEM((16,), jnp.float32),
        ],
        compiler_params=pltpu.CompilerParams(needs_layout_passes=False),
    )
    lov = sc_fn(keys_flat)

    mse = jnp.sum(msep) / (B * 3 * H * W)
    return mse + jnp.sum(lov[:, 0]) / B


# speedup vs baseline: 10.2097x; 1.5353x over previous
import jax
import jax.numpy as jnp
import numpy as np
from jax import lax
from jax.experimental import pallas as pl
from jax.experimental.pallas import tpu as pltpu
from jax.experimental.pallas import tpu_sc as plsc

B, H, W = 4, 512, 512
N = H * W
NPAIR = 20
NTILE = 16
CHUNK = N // NTILE
SUB = CHUNK // 16
NBANK = 4
QSUB = SUB // NBANK
QTILE = CHUNK // NBANK
RAD = 256
NC_TC = 8
MIN32 = np.int32(-2147483648)


def _prep_body(d0, d1, d2, d3, d4, gt, rgl, rgo, kall, msep):
    gti = gt[...]
    gtf = gti.astype(jnp.float32)
    sign = 2.0 * gtf - 1.0
    for dd, dref in enumerate((d0, d1, d2, d3, d4)):
        x = dref[...]
        e = 1.0 - x * sign
        bi = lax.bitcast_convert_type(e, jnp.int32)
        srt = jnp.where(bi < 0, jnp.bitwise_xor(bi, -1),
                        jnp.bitwise_xor(bi, MIN32))
        k = jnp.bitwise_xor(srt, -1)
        k = jnp.bitwise_or(jnp.bitwise_and(k, -2), gti)
        kall[dd] = k
    g3 = gtf[:, None]
    df = (rgo[...] - rgl[...]) * g3
    msep[0, 0, 0] = jnp.sum(df * df)


_GDN = lax.GatherDimensionNumbers(offset_dims=(), collapsed_slice_dims=(0,),
                                  start_index_map=(0,))


def _lane_bcast(v, idx):
    return lax.gather(v, idx[:, None], dimension_numbers=_GDN,
                      slice_sizes=(1,),
                      mode=lax.GatherScatterMode.PROMISE_IN_BOUNDS)


def _sc_body(keys_hbm, out_hbm, bufA, bufB, gT, gLab, gRed,
             tk, h0, h1, h2, h3, b0, b1, b2, b3, g0, g1, g2, g3,
             gtl, glabv, gredv, posb, stg, stgi, stgf):
    c = lax.axis_index("c")
    s = lax.axis_index("s")
    iota = lax.iota(jnp.int32, 16)
    iota_sub = iota * SUB
    iota_rad = iota * RAD
    ones = jnp.ones((16,), jnp.int32)
    zer16 = jnp.zeros((16,), jnp.int32)
    hists = (h0, h1, h2, h3)
    bass = (b0, b1, b2, b3)
    dgbs = (g0, g1, g2, g3)

    def z0(i, _):
        for hk in hists:
            plsc.store_scatter(hk, [i * 16 + iota], zer16)
        return 0
    lax.fori_loop(0, RAD * 16 // 16, z0, 0)

    def do_pair(jp, carry_outer):
        p = 2 * jp + c
        base_off = p * N

        for pass_i in range(4):
            if pass_i == 0:
                off = pl.multiple_of(base_off + s * CHUNK, CHUNK)
                pltpu.sync_copy(keys_hbm.at[pl.ds(off, CHUNK)], tk)
                dst = bufB
            else:
                src = bufB if pass_i % 2 == 1 else bufA
                dst = bufA if pass_i % 2 == 1 else bufB
                off = pl.multiple_of(s * CHUNK, CHUNK)
                pltpu.sync_copy(src.at[pl.ds(off, CHUNK)], tk)
            shift = jnp.full((16,), 8 * pass_i, jnp.int32)

            @plsc.parallel_loop(0, QSUB, unroll=4)
            def _(jj):
                for k in range(NBANK):
                    idxv = jj * 64 + k * 16 + iota
                    kv = plsc.load_gather(tk, [idxv])
                    dg = jnp.bitwise_and(
                        lax.shift_right_logical(kv, shift), 255)
                    fl = iota_rad + dg
                    plsc.addupdate_scatter(hists[k], [fl], ones)
                    plsc.store_scatter(dgbs[k], [jj * 16 + iota], fl)

            @plsc.parallel_loop(0, 16, unroll=2)
            def _(g):
                gbase = g * 16 + iota
                tot = zer16
                for l in range(16):
                    for k in range(NBANK):
                        tot = tot + plsc.load_gather(
                            hists[k], [l * RAD + gbase])
                plsc.store_scatter(stg, [gbase], tot)
            pltpu.sync_copy(stg, gT.at[s])
            plsc.subcore_barrier()
            pltpu.sync_copy(gT, gtl)

            def sb(g, carry):
                gbase = g * 16 + iota
                tots = zer16
                pre = zer16
                for t in range(16):
                    row = plsc.load_gather(
                        gtl, [jnp.full((16,), t, jnp.int32), gbase])
                    tots = tots + row
                    pre = pre + jnp.where(t < s, row, 0)
                ex = plsc.cumsum(tots) - tots
                base_dig = carry + pre + ex
                hacc = zer16
                for k in range(NBANK):
                    for l in range(16):
                        hseg = plsc.load_gather(hists[k], [l * RAD + gbase])
                        plsc.store_scatter(bass[k], [l * RAD + gbase],
                                           base_dig + hacc)
                        hacc = hacc + hseg
                        plsc.store_scatter(hists[k], [l * RAD + gbase],
                                           zer16)
                return carry + jnp.sum(tots)
            lax.fori_loop(0, 16, sb, jnp.int32(0))

            def pb(jj, _):
                fls = [plsc.load_gather(dgbs[k], [jj * 16 + iota])
                       for k in range(NBANK)]
                for k in range(NBANK):
                    pos = plsc.load_gather(bass[k], [fls[k]])
                    plsc.store_scatter(bass[k], [fls[k]], pos + ones)
                    q = jnp.bitwise_and(pos, CHUNK - 1)
                    tb = pos - q
                    p2 = (tb + jnp.bitwise_and(q, 255) * 64
                          + jnp.right_shift(q, 8))
                    plsc.store_scatter(posb, [jj * 64 + k * 16 + iota], p2)
                return 0
            lax.fori_loop(0, QSUB, pb, 0, unroll=4)
            pltpu.sync_copy(tk, dst.at[posb])
            plsc.subcore_barrier()

        off = pl.multiple_of(s * CHUNK, CHUNK)
        pltpu.sync_copy(bufA.at[pl.ds(off, CHUNK)], tk)

        def l1(jj, accs):
            out = []
            for k in range(NBANK):
                kv = plsc.load_gather(tk, [jj * 64 + k * 16 + iota])
                out.append(accs[k] + jnp.bitwise_and(kv, 1))
            return tuple(out)
        accs = lax.fori_loop(0, QSUB, l1, (zer16,) * NBANK, unroll=4)
        tot = (jnp.sum(accs[0]) + jnp.sum(accs[1]) + jnp.sum(accs[2])
               + jnp.sum(accs[3]))
        stgi[...] = jnp.full((16,), tot, jnp.int32)
        pltpu.sync_copy(stgi, gLab.at[s])
        plsc.subcore_barrier()
        pltpu.sync_copy(gLab, glabv)
        totv = plsc.load_gather(glabv, [iota, zer16])
        pcount = jnp.sum(totv)
        cbase = jnp.sum(jnp.where(iota == s, plsc.cumsum(totv) - totv, 0))
        pf = pcount.astype(jnp.float32)
        ftn = jnp.full((16,), 15, jnp.int32)
        cbs = []
        running = cbase
        for k in range(NBANK):
            inc = plsc.cumsum(accs[k])
            cbs.append(running + inc - accs[k])
            running = running + _lane_bcast(inc, ftn)

        def l2(jj, carry):
            cruns, laccs = carry
            ncr, nla = [], []
            for k in range(NBANK):
                kv = plsc.load_gather(tk, [jj * 64 + k * 16 + iota])
                lab = jnp.bitwise_and(kv, 1)
                s2 = jnp.bitwise_xor(kv, -1)
                u2 = jnp.where(s2 < 0, jnp.bitwise_xor(s2, MIN32),
                               jnp.bitwise_xor(s2, -1))
                e = plsc.bitcast(u2, jnp.float32)
                relu = jnp.maximum(e, 0.0)
                cvec = (cbs[k] + cruns[k] + lab).astype(jnp.float32)
                ivec = (s * CHUNK + (k * 16 + iota) * QSUB + jj
                        ).astype(jnp.float32)
                t2 = ivec + pf - cvec
                ispos = lab > 0
                bad = jnp.logical_and(t2 < 0.5, jnp.logical_not(ispos))
                numr = jnp.where(ispos, 1.0, pf - cvec)
                numr = jnp.where(bad, 1.0, numr)
                den = jnp.where(ispos, t2 + 1.0, (t2 + 1.0) * t2)
                den = jnp.where(bad, 1.0, den)
                nla.append(laccs[k] + relu * numr / den)
                ncr.append(cruns[k] + lab)
            return (tuple(ncr), tuple(nla))

        zf = jnp.zeros((16,), jnp.float32)
        _, laccs = lax.fori_loop(0, QSUB, l2,
                                 ((zer16,) * NBANK, (zf,) * NBANK),
                                 unroll=4)
        part = jnp.sum(laccs[0] + laccs[1] + laccs[2] + laccs[3])
        stgf[...] = jnp.full((16,), part, jnp.float32)
        pltpu.sync_copy(stgf, gRed.at[s])
        plsc.subcore_barrier()

        @pl.when(s == 0)
        def _():
            pltpu.sync_copy(gRed, gredv)
            pv = plsc.load_gather(gredv, [iota, zer16])
            stgf[...] = jnp.full((16,), jnp.sum(pv), jnp.float32)
            pltpu.sync_copy(stgf, out_hbm.at[p])
        plsc.subcore_barrier()
        return carry_outer

    lax.fori_loop(0, NPAIR // 2, do_pair, 0)


def kernel(d0, d1, d2, d3, d4, r_glass, r_global, ground_truth):
    dflat = [x.reshape(B, H, W) for x in (d0, d1, d2, d3, d4)]
    gt2 = ground_truth.reshape(B, H, W)
    rgl = r_glass.reshape(B, 3, H, W)
    rgo = r_global.reshape(B, 3, H, W)

    ROWS = H // NC_TC
    lin = pl.BlockSpec((1, ROWS, W), lambda i, j: (i, j, 0))
    rin = pl.BlockSpec((1, 3, ROWS, W), lambda i, j: (i, 0, j, 0))
    kall, msep = pl.pallas_call(
        _prep_body,
        grid=(B, NC_TC),
        in_specs=[lin] * 5 + [lin, rin, rin],
        out_specs=[pl.BlockSpec((5, 1, ROWS, W), lambda i, j: (0, i, j, 0)),
                   pl.BlockSpec((1, 1, 1), lambda i, j: (i * NC_TC + j, 0, 0),
                                memory_space=pltpu.SMEM)],
        out_shape=[jax.ShapeDtypeStruct((5, B, H, W), jnp.int32),
                   jax.ShapeDtypeStruct((B * NC_TC, 1, 1), jnp.float32)],
    )(*dflat, gt2, rgl, rgo)

    keys_flat = kall.reshape(NPAIR * N)

    mesh = plsc.VectorSubcoreMesh(core_axis_name="c", subcore_axis_name="s")
    sc_fn = pl.kernel(
        _sc_body,
        out_type=jax.ShapeDtypeStruct((NPAIR, 16), jnp.float32),
        mesh=mesh,
        scratch_types=[
            pltpu.VMEM_SHARED((N,), jnp.int32),
            pltpu.VMEM_SHARED((N,), jnp.int32),
            pltpu.VMEM_SHARED((16, RAD), jnp.int32),
            pltpu.VMEM_SHARED((16, 16), jnp.int32),
            pltpu.VMEM_SHARED((16, 16), jnp.float32),
            pltpu.VMEM((CHUNK,), jnp.int32),
            pltpu.VMEM((RAD * 16,), jnp.int32),
            pltpu.VMEM((RAD * 16,), jnp.int32),
            pltpu.VMEM((RAD * 16,), jnp.int32),
            pltpu.VMEM((RAD * 16,), jnp.int32),
            pltpu.VMEM((RAD * 16,), jnp.int32),
            pltpu.VMEM((RAD * 16,), jnp.int32),
            pltpu.VMEM((RAD * 16,), jnp.int32),
            pltpu.VMEM((RAD * 16,), jnp.int32),
            pltpu.VMEM((QSUB * 16,), jnp.int32),
            pltpu.VMEM((QSUB * 16,), jnp.int32),
            pltpu.VMEM((QSUB * 16,), jnp.int32),
            pltpu.VMEM((QSUB * 16,), jnp.int32),
            pltpu.VMEM((16, RAD), jnp.int32),
            pltpu.VMEM((16, 16), jnp.int32),
            pltpu.VMEM((16, 16), jnp.float32),
            pltpu.VMEM((CHUNK,), jnp.int32),
            pltpu.VMEM((RAD,), jnp.int32),
            pltpu.VMEM((16,), jnp.int32),
            pltpu.VMEM((16,), jnp.float32),
        ],
        compiler_params=pltpu.CompilerParams(needs_layout_passes=False),
    )
    lov = sc_fn(keys_flat)

    mse = jnp.sum(msep) / (B * 3 * H * W)
    return mse + jnp.sum(lov[:, 0]) / B


# speedup vs baseline: 10.3518x; 1.0139x over previous
import jax
import jax.numpy as jnp
import numpy as np
from jax import lax
from jax.experimental import pallas as pl
from jax.experimental.pallas import tpu as pltpu
from jax.experimental.pallas import tpu_sc as plsc

B, H, W = 4, 512, 512
N = H * W
NPAIR = 20
NTILE = 16
CHUNK = N // NTILE
SUB = CHUNK // 16
NBANK = 4
QSUB = SUB // NBANK
QTILE = CHUNK // NBANK
RAD = 256
NC_TC = 8
MIN32 = np.int32(-2147483648)


def _prep_body(d0, d1, d2, d3, d4, gt, rgl, rgo, kall, msep):
    gti = gt[...]
    gtf = gti.astype(jnp.float32)
    sign = 2.0 * gtf - 1.0
    for dd, dref in enumerate((d0, d1, d2, d3, d4)):
        x = dref[...]
        e = 1.0 - x * sign
        bi = lax.bitcast_convert_type(e, jnp.int32)
        srt = jnp.where(bi < 0, jnp.bitwise_xor(bi, -1),
                        jnp.bitwise_xor(bi, MIN32))
        k = jnp.bitwise_xor(srt, -1)
        k = jnp.bitwise_or(jnp.bitwise_and(k, -2), gti)
        kall[dd] = k
    g3 = gtf[:, None]
    df = (rgo[...] - rgl[...]) * g3
    msep[0, 0, 0] = jnp.sum(df * df)


_GDN = lax.GatherDimensionNumbers(offset_dims=(), collapsed_slice_dims=(0,),
                                  start_index_map=(0,))


def _lane_bcast(v, idx):
    return lax.gather(v, idx[:, None], dimension_numbers=_GDN,
                      slice_sizes=(1,),
                      mode=lax.GatherScatterMode.PROMISE_IN_BOUNDS)


def _sc_body(keys_hbm, out_hbm, bufA, bufB, gT, gLab, gRed,
             tk, h0, h1, h2, h3, b0, b1, b2, b3, g0, g1, g2, g3,
             gtl, glabv, gredv, posb, stg, stgi, stgf):
    c = lax.axis_index("c")
    s = lax.axis_index("s")
    iota = lax.iota(jnp.int32, 16)
    iota_sub = iota * SUB
    iota_rad = iota * RAD
    ones = jnp.ones((16,), jnp.int32)
    zer16 = jnp.zeros((16,), jnp.int32)
    hists = (h0, h1, h2, h3)
    bass = (b0, b1, b2, b3)
    dgbs = (g0, g1, g2, g3)

    def z0(i, _):
        for hk in hists:
            plsc.store_scatter(hk, [i * 16 + iota], zer16)
        return 0
    lax.fori_loop(0, RAD * 16 // 16, z0, 0)

    def do_pair(jp, carry_outer):
        p = 2 * jp + c
        base_off = p * N

        for pass_i in range(4):
            if pass_i == 0:
                off = pl.multiple_of(base_off + s * CHUNK, CHUNK)
                pltpu.sync_copy(keys_hbm.at[pl.ds(off, CHUNK)], tk)
                dst = bufB
            else:
                src = bufB if pass_i % 2 == 1 else bufA
                dst = bufA if pass_i % 2 == 1 else bufB
                off = pl.multiple_of(s * CHUNK, CHUNK)
                pltpu.sync_copy(src.at[pl.ds(off, CHUNK)], tk)
            shift = jnp.full((16,), 8 * pass_i, jnp.int32)

            @plsc.parallel_loop(0, QSUB, unroll=4)
            def _(jj):
                for k in range(NBANK):
                    kv = tk[pl.ds(jj * 64 + k * 16, 16)]
                    dg = jnp.bitwise_and(
                        lax.shift_right_logical(kv, shift), 255)
                    fl = iota_rad + dg
                    plsc.addupdate_scatter(hists[k], [fl], ones)
                    dgbs[k][pl.ds(jj * 16, 16)] = fl

            @plsc.parallel_loop(0, 16, unroll=2)
            def _(g):
                gbase = g * 16 + iota
                tot = zer16
                for l in range(16):
                    for k in range(NBANK):
                        tot = tot + plsc.load_gather(
                            hists[k], [l * RAD + gbase])
                plsc.store_scatter(stg, [gbase], tot)
            pltpu.sync_copy(stg, gT.at[s])
            plsc.subcore_barrier()
            pltpu.sync_copy(gT, gtl)

            def sb(g, carry):
                gbase = g * 16 + iota
                tots = zer16
                pre = zer16
                for t in range(16):
                    row = plsc.load_gather(
                        gtl, [jnp.full((16,), t, jnp.int32), gbase])
                    tots = tots + row
                    pre = pre + jnp.where(t < s, row, 0)
                ex = plsc.cumsum(tots) - tots
                base_dig = carry + pre + ex
                hacc = zer16
                for k in range(NBANK):
                    for l in range(16):
                        hseg = plsc.load_gather(hists[k], [l * RAD + gbase])
                        plsc.store_scatter(bass[k], [l * RAD + gbase],
                                           base_dig + hacc)
                        hacc = hacc + hseg
                        plsc.store_scatter(hists[k], [l * RAD + gbase],
                                           zer16)
                return carry + jnp.sum(tots)
            lax.fori_loop(0, 16, sb, jnp.int32(0))

            def pb(jj, _):
                fls = [dgbs[k][pl.ds(jj * 16, 16)] for k in range(NBANK)]
                for k in range(NBANK):
                    pos = plsc.load_gather(bass[k], [fls[k]])
                    plsc.store_scatter(bass[k], [fls[k]], pos + ones)
                    q = jnp.bitwise_and(pos, CHUNK - 1)
                    tb = pos - q
                    p2 = (tb + jnp.bitwise_and(q, 255) * 64
                          + jnp.right_shift(q, 8))
                    posb[pl.ds(jj * 64 + k * 16, 16)] = p2
                return 0
            lax.fori_loop(0, QSUB, pb, 0, unroll=4)
            pltpu.sync_copy(tk, dst.at[posb])
            plsc.subcore_barrier()

        off = pl.multiple_of(s * CHUNK, CHUNK)
        pltpu.sync_copy(bufA.at[pl.ds(off, CHUNK)], tk)

        def l1(jj, accs):
            out = []
            for k in range(NBANK):
                kv = tk[pl.ds(jj * 64 + k * 16, 16)]
                out.append(accs[k] + jnp.bitwise_and(kv, 1))
            return tuple(out)
        accs = lax.fori_loop(0, QSUB, l1, (zer16,) * NBANK, unroll=4)
        tot = (jnp.sum(accs[0]) + jnp.sum(accs[1]) + jnp.sum(accs[2])
               + jnp.sum(accs[3]))
        stgi[...] = jnp.full((16,), tot, jnp.int32)
        pltpu.sync_copy(stgi, gLab.at[s])
        plsc.subcore_barrier()
        pltpu.sync_copy(gLab, glabv)
        totv = plsc.load_gather(glabv, [iota, zer16])
        pcount = jnp.sum(totv)
        cbase = jnp.sum(jnp.where(iota == s, plsc.cumsum(totv) - totv, 0))
        pf = pcount.astype(jnp.float32)
        ftn = jnp.full((16,), 15, jnp.int32)
        cbs = []
        running = cbase
        for k in range(NBANK):
            inc = plsc.cumsum(accs[k])
            cbs.append(running + inc - accs[k])
            running = running + _lane_bcast(inc, ftn)

        def l2(jj, carry):
            cruns, laccs = carry
            ncr, nla = [], []
            for k in range(NBANK):
                kv = tk[pl.ds(jj * 64 + k * 16, 16)]
                lab = jnp.bitwise_and(kv, 1)
                s2 = jnp.bitwise_xor(kv, -1)
                u2 = jnp.where(s2 < 0, jnp.bitwise_xor(s2, MIN32),
                               jnp.bitwise_xor(s2, -1))
                e = plsc.bitcast(u2, jnp.float32)
                relu = jnp.maximum(e, 0.0)
                cvec = (cbs[k] + cruns[k] + lab).astype(jnp.float32)
                ivec = (s * CHUNK + (k * 16 + iota) * QSUB + jj
                        ).astype(jnp.float32)
                t2 = ivec + pf - cvec
                ispos = lab > 0
                bad = jnp.logical_and(t2 < 0.5, jnp.logical_not(ispos))
                numr = jnp.where(ispos, 1.0, pf - cvec)
                numr = jnp.where(bad, 1.0, numr)
                den = jnp.where(ispos, t2 + 1.0, (t2 + 1.0) * t2)
                den = jnp.where(bad, 1.0, den)
                nla.append(laccs[k] + relu * numr / den)
                ncr.append(cruns[k] + lab)
            return (tuple(ncr), tuple(nla))

        zf = jnp.zeros((16,), jnp.float32)
        _, laccs = lax.fori_loop(0, QSUB, l2,
                                 ((zer16,) * NBANK, (zf,) * NBANK),
                                 unroll=4)
        part = jnp.sum(laccs[0] + laccs[1] + laccs[2] + laccs[3])
        stgf[...] = jnp.full((16,), part, jnp.float32)
        pltpu.sync_copy(stgf, gRed.at[s])
        plsc.subcore_barrier()

        @pl.when(s == 0)
        def _():
            pltpu.sync_copy(gRed, gredv)
            pv = plsc.load_gather(gredv, [iota, zer16])
            stgf[...] = jnp.full((16,), jnp.sum(pv), jnp.float32)
            pltpu.sync_copy(stgf, out_hbm.at[p])
        plsc.subcore_barrier()
        return carry_outer

    lax.fori_loop(0, NPAIR // 2, do_pair, 0)


def kernel(d0, d1, d2, d3, d4, r_glass, r_global, ground_truth):
    dflat = [x.reshape(B, H, W) for x in (d0, d1, d2, d3, d4)]
    gt2 = ground_truth.reshape(B, H, W)
    rgl = r_glass.reshape(B, 3, H, W)
    rgo = r_global.reshape(B, 3, H, W)

    ROWS = H // NC_TC
    lin = pl.BlockSpec((1, ROWS, W), lambda i, j: (i, j, 0))
    rin = pl.BlockSpec((1, 3, ROWS, W), lambda i, j: (i, 0, j, 0))
    kall, msep = pl.pallas_call(
        _prep_body,
        grid=(B, NC_TC),
        in_specs=[lin] * 5 + [lin, rin, rin],
        out_specs=[pl.BlockSpec((5, 1, ROWS, W), lambda i, j: (0, i, j, 0)),
                   pl.BlockSpec((1, 1, 1), lambda i, j: (i * NC_TC + j, 0, 0),
                                memory_space=pltpu.SMEM)],
        out_shape=[jax.ShapeDtypeStruct((5, B, H, W), jnp.int32),
                   jax.ShapeDtypeStruct((B * NC_TC, 1, 1), jnp.float32)],
    )(*dflat, gt2, rgl, rgo)

    keys_flat = kall.reshape(NPAIR * N)

    mesh = plsc.VectorSubcoreMesh(core_axis_name="c", subcore_axis_name="s")
    sc_fn = pl.kernel(
        _sc_body,
        out_type=jax.ShapeDtypeStruct((NPAIR, 16), jnp.float32),
        mesh=mesh,
        scratch_types=[
            pltpu.VMEM_SHARED((N,), jnp.int32),
            pltpu.VMEM_SHARED((N,), jnp.int32),
            pltpu.VMEM_SHARED((16, RAD), jnp.int32),
            pltpu.VMEM_SHARED((16, 16), jnp.int32),
            pltpu.VMEM_SHARED((16, 16), jnp.float32),
            pltpu.VMEM((CHUNK,), jnp.int32),
            pltpu.VMEM((RAD * 16,), jnp.int32),
            pltpu.VMEM((RAD * 16,), jnp.int32),
            pltpu.VMEM((RAD * 16,), jnp.int32),
            pltpu.VMEM((RAD * 16,), jnp.int32),
            pltpu.VMEM((RAD * 16,), jnp.int32),
            pltpu.VMEM((RAD * 16,), jnp.int32),
            pltpu.VMEM((RAD * 16,), jnp.int32),
            pltpu.VMEM((RAD * 16,), jnp.int32),
            pltpu.VMEM((QSUB * 16,), jnp.int32),
            pltpu.VMEM((QSUB * 16,), jnp.int32),
            pltpu.VMEM((QSUB * 16,), jnp.int32),
            pltpu.VMEM((QSUB * 16,), jnp.int32),
            pltpu.VMEM((16, RAD), jnp.int32),
            pltpu.VMEM((16, 16), jnp.int32),
            pltpu.VMEM((16, 16), jnp.float32),
            pltpu.VMEM((CHUNK,), jnp.int32),
            pltpu.VMEM((RAD,), jnp.int32),
            pltpu.VMEM((16,), jnp.int32),
            pltpu.VMEM((16,), jnp.float32),
        ],
        compiler_params=pltpu.CompilerParams(needs_layout_passes=False),
    )
    lov = sc_fn(keys_flat)

    mse = jnp.sum(msep) / (B * 3 * H * W)
    return mse + jnp.sum(lov[:, 0]) / B
